# baseline (speedup 1.0000x reference)
import jax
import jax.numpy as jnp
from jax.experimental import pallas as pl
from jax.experimental.pallas import tpu as pltpu

HIGH = jax.lax.Precision.HIGHEST


def _fused_kernel(hw, bt, x_ref, wb_ref, bb_ref, wfe_ref, bfe_ref,
                  wcl_ref, bcl_ref, emb_ref, met_ref, clu_ref):
    x = x_ref[...]

    emb_ref[...] = jnp.dot(x.astype(jnp.bfloat16), wb_ref[...],
                           preferred_element_type=jnp.float32) + bb_ref[...]

    mt = x.shape[0]
    row_b = jax.lax.broadcasted_iota(jnp.int32, (bt, mt), 1) // hw
    tgt_b = jax.lax.broadcasted_iota(jnp.int32, (bt, mt), 0)
    pool = jnp.where(row_b == tgt_b, jnp.float32(1.0), jnp.float32(0.0))
    x_mean = jnp.dot(pool, x, preferred_element_type=jnp.float32) * (1.0 / hw)

    feats = jnp.dot(x_mean, wfe_ref[...],
                    preferred_element_type=jnp.float32) + bfe_ref[...]
    inv_f = jax.lax.rsqrt(
        jnp.maximum(jnp.sum(feats * feats, axis=-1, keepdims=True), 1e-24))
    metric = feats * inv_f

    cluster = jnp.dot(metric, wcl_ref[...],
                      preferred_element_type=jnp.float32) + bcl_ref[...]
    inv_c = jax.lax.rsqrt(
        jnp.maximum(jnp.sum(cluster * cluster, axis=-1, keepdims=True), 1e-24))

    met_ref[...] = metric
    clu_ref[...] = cluster * inv_c


def kernel(x_nchw, w_base, b_base, w_feat, b_feat, bn_gamma, bn_beta,
           bn_rm, bn_rv, w_cl, b_cl):
    B, Cin, H, W = x_nchw.shape
    HW = H * W
    Cemb = w_base.shape[1]
    low_dim = w_feat.shape[1]
    n_cluster = w_cl.shape[1]

    bt = B
    for cand in (16, 8, 32, 40, 80):
        if B % cand == 0 and (cand * HW) % 8 == 0:
            bt = cand
            break
    n_tiles = B // bt
    Mt = bt * HW

    x2d = jnp.transpose(x_nchw.reshape(B, Cin, HW), (0, 2, 1)).reshape(B * HW, Cin)
    wb_bf = w_base.astype(jnp.bfloat16)

    w_feat_eff = jnp.dot(w_base, w_feat, precision=HIGH)
    b_feat_eff = jnp.dot(b_base, w_feat, precision=HIGH) + b_feat
    s = bn_gamma * jax.lax.rsqrt(bn_rv + 1e-5)
    w_cl_eff = w_cl * s.reshape(low_dim, 1)
    b_cl_eff = b_cl + jnp.dot(bn_beta - bn_rm * s, w_cl, precision=HIGH)

    flops = 2 * B * HW * Cin * Cemb + 2 * B * Cin * low_dim \
        + 2 * B * low_dim * n_cluster
    bytes_accessed = 4 * (B * HW * Cin + Cin * low_dim + low_dim
                          + low_dim * n_cluster + n_cluster
                          + B * HW * Cemb + B * (low_dim + n_cluster)) \
        + 2 * Cin * Cemb

    body = lambda *refs: _fused_kernel(HW, bt, *refs)
    emb2d, metric, cluster_n = pl.pallas_call(
        body,
        out_shape=(
            jax.ShapeDtypeStruct((B * HW, Cemb), jnp.float32),
            jax.ShapeDtypeStruct((B, low_dim), jnp.float32),
            jax.ShapeDtypeStruct((B, n_cluster), jnp.float32),
        ),
        grid=(n_tiles,),
        in_specs=[
            pl.BlockSpec((Mt, Cin), lambda i: (i, 0)),
            pl.BlockSpec((Cin, Cemb), lambda i: (0, 0)),
            pl.BlockSpec((1, Cemb), lambda i: (0, 0)),
            pl.BlockSpec((Cin, low_dim), lambda i: (0, 0)),
            pl.BlockSpec((1, low_dim), lambda i: (0, 0)),
            pl.BlockSpec((low_dim, n_cluster), lambda i: (0, 0)),
            pl.BlockSpec((1, n_cluster), lambda i: (0, 0)),
        ],
        out_specs=(
            pl.BlockSpec((Mt, Cemb), lambda i: (i, 0)),
            pl.BlockSpec((bt, low_dim), lambda i: (i, 0)),
            pl.BlockSpec((bt, n_cluster), lambda i: (i, 0)),
        ),
        compiler_params=pltpu.CompilerParams(dimension_semantics=("parallel",)),
        cost_estimate=pl.CostEstimate(flops=flops, transcendentals=4 * B,
                                      bytes_accessed=bytes_accessed),
    )(x2d, wb_bf, b_base, w_feat_eff, b_feat_eff, w_cl_eff, b_cl_eff)

    emb7x7 = jnp.transpose(
        emb2d.reshape(B, HW, Cemb), (0, 2, 1)).reshape(B, Cemb, H, W)
    return metric, cluster_n, emb7x7
